# baseline (speedup 1.0000x reference)
import functools

import jax
import jax.numpy as jnp
from jax import lax
from jax.experimental import pallas as pl
from jax.experimental.pallas import tpu as pltpu
from jax.experimental.pallas import tpu_sc as plsc

N = 10000
DEG = 16
E = N * DEG
D = 256
D2 = 2 * D
ALPHA = 0.2

NC = 2
NS = 16
L = 16
NW = NC * NS

NP = 10240
EP = NP * DEG
NODES_PER_W = NP // NW
BN = 8
NODES_C0 = 448
NODES_C1 = 192
NBLK0 = NODES_C0 // BN
NBLK1 = NODES_C1 // BN
NBLK = NBLK0
NBLKG = NP // BN
BE = BN * DEG

DI = D2 // 2
GCH = DI // L


def _leaky(v):
    return jnp.where(v >= 0, v, ALPHA * v)


def _mesh():
    return plsc.VectorSubcoreMesh(
        core_axis_name="c", subcore_axis_name="s", num_cores=NC, num_subcores=NS
    )


def _wid():
    return lax.axis_index("s") * NC + lax.axis_index("c")


def _worker_span():
    c = lax.axis_index("c")
    s = lax.axis_index("s")
    nbase = jnp.where(c == 0, s * NODES_C0, NS * NODES_C0 + s * NODES_C1)
    nblk = jnp.where(c == 0, NBLK0, NBLK1)
    return pl.multiple_of(nbase, 64), nblk


def _lohi(v):
    lo = plsc.bitcast(jnp.left_shift(v, 16), jnp.float32)
    hi = plsc.bitcast(jnp.bitwise_and(v, jnp.int32(-65536)), jnp.float32)
    return lo, hi


def _pack_bf(lo, hi):
    bl = lax.shift_right_logical(plsc.bitcast(lo, jnp.int32) + 0x8000, 16)
    bh = jnp.bitwise_and(plsc.bitcast(hi, jnp.int32) + 0x8000, jnp.int32(-65536))
    return jnp.bitwise_or(bl, bh)


_TC_BLK = 1024


def _tc_pack(h):
    lo = h[:, : D // 2]
    hi = h[:, D // 2 :]
    bl = lax.shift_right_logical(lax.bitcast_convert_type(lo, jnp.int32) + 0x8000, 16)
    bh = jnp.bitwise_and(lax.bitcast_convert_type(hi, jnp.int32) + 0x8000, jnp.int32(-65536))
    return jnp.bitwise_or(bl, bh)


def _tc_body(x_ref, wh_ref, wl_ref, a1h_ref, a2h_ref, a1l_ref, a2l_ref, hcat_ref, sh_ref, th_ref, sl_ref, tl_ref):
    xb = x_ref[...]
    hh = jnp.dot(xb, wh_ref[...], preferred_element_type=jnp.float32)
    hl = jnp.dot(xb, wl_ref[...], preferred_element_type=jnp.float32)
    hcat_ref[:, : D // 2] = _tc_pack(hh)
    hcat_ref[:, D // 2 :] = _tc_pack(hl)
    sh_ref[...] = jnp.dot(hh, a1h_ref[...], preferred_element_type=jnp.float32)
    th_ref[...] = jnp.dot(hh, a2h_ref[...], preferred_element_type=jnp.float32)
    sl_ref[...] = jnp.dot(hl, a1l_ref[...], preferred_element_type=jnp.float32)
    tl_ref[...] = jnp.dot(hl, a2l_ref[...], preferred_element_type=jnp.float32)


def _tc_stage(xp, W_high, W_low, a1h, a2h, a1l, a2l):
    vec = pl.BlockSpec((D,), lambda i: (0,))
    row = pl.BlockSpec((_TC_BLK,), lambda i: (i,))
    return pl.pallas_call(
        _tc_body,
        grid=(NP // _TC_BLK,),
        in_specs=[
            pl.BlockSpec((_TC_BLK, D), lambda i: (i, 0)),
            pl.BlockSpec((D, D), lambda i: (0, 0)),
            pl.BlockSpec((D, D), lambda i: (0, 0)),
            vec,
            vec,
            vec,
            vec,
        ],
        out_specs=[
            pl.BlockSpec((_TC_BLK, DI), lambda i: (i, 0)),
            row,
            row,
            row,
            row,
        ],
        out_shape=[
            jax.ShapeDtypeStruct((NP, DI), jnp.int32),
            jax.ShapeDtypeStruct((NP,), jnp.float32),
            jax.ShapeDtypeStruct((NP,), jnp.float32),
            jax.ShapeDtypeStruct((NP,), jnp.float32),
            jax.ShapeDtypeStruct((NP,), jnp.float32),
        ],
    )(xp, W_high, W_low, a1h, a2h, a1l, a2l)


def _stage2(hcat, se_h, t_h, se_l, t_l, dst2d):
    @functools.partial(
        pl.kernel,
        mesh=_mesh(),
        compiler_params=pltpu.CompilerParams(needs_layout_passes=False),
        out_type=[
            jax.ShapeDtypeStruct((NP, DI), jnp.int32),
            jax.ShapeDtypeStruct((NBLKG, BE), jnp.float32),
            jax.ShapeDtypeStruct((NBLKG, BE), jnp.float32),
        ],
        scratch_types=[
            pltpu.VMEM((NBLK, BE), jnp.int32),
            pltpu.VMEM((NBLK, BE), jnp.float32),
            pltpu.VMEM((NBLK, BE), jnp.float32),
            pltpu.VMEM((NBLK, BE), jnp.float32),
            pltpu.VMEM((NBLK, BE), jnp.float32),
            pltpu.VMEM((BE, DI), jnp.int32),
            pltpu.VMEM((BE, DI), jnp.int32),
            pltpu.VMEM((BE,), jnp.float32),
            pltpu.VMEM((BE,), jnp.float32),
            pltpu.VMEM((BE,), jnp.float32),
            pltpu.VMEM((BE,), jnp.float32),
            pltpu.VMEM((BN, DI), jnp.int32),
            pltpu.VMEM((BN, DI), jnp.int32),
            pltpu.VMEM((BN, DI), jnp.int32),
            pltpu.VMEM((BN, DI), jnp.int32),
            pltpu.SemaphoreType.DMA,
            pltpu.SemaphoreType.DMA,
            pltpu.SemaphoreType.DMA,
            pltpu.SemaphoreType.DMA,
            pltpu.SemaphoreType.DMA,
            pltpu.SemaphoreType.DMA,
            pltpu.SemaphoreType.DMA,
            pltpu.SemaphoreType.DMA,
            pltpu.SemaphoreType.DMA,
            pltpu.SemaphoreType.DMA,
        ],
    )
    def k(
        hcat_hbm,
        sh_hbm,
        th_hbm,
        sl_hbm,
        tl_hbm,
        dst2d_hbm,
        hacat_hbm,
        wph_hbm,
        wpl_hbm,
        idx2d,
        seh_v,
        sel_v,
        wph_v,
        wpl_v,
        g0,
        g1,
        tvh0,
        tvh1,
        tvl0,
        tvl1,
        o0,
        o1,
        agg0,
        agg1,
        sg0,
        sg1,
        sth0,
        sth1,
        stl0,
        stl1,
        so0,
        so1,
        sout0,
        sout1,
    ):
        gbuf = (g0, g1)
        tvh = (tvh0, tvh1)
        tvl = (tvl0, tvl1)
        obuf = (o0, o1)
        aggbuf = (agg0, agg1)
        sem_g = (sg0, sg1)
        sem_th = (sth0, sth1)
        sem_tl = (stl0, stl1)
        sem_o = (so0, so1)
        sem_out = (sout0, sout1)

        nbase, nblk = _worker_span()
        gbase = pl.multiple_of(nbase // BN, 8)
        _R = NBLK0 - NBLK1
        pltpu.sync_copy(dst2d_hbm.at[pl.ds(gbase, NBLK1)], idx2d.at[pl.ds(0, NBLK1)])
        pltpu.sync_copy(sh_hbm.at[pl.ds(gbase, NBLK1)], seh_v.at[pl.ds(0, NBLK1)])
        pltpu.sync_copy(sl_hbm.at[pl.ds(gbase, NBLK1)], sel_v.at[pl.ds(0, NBLK1)])

        @pl.when(nblk > NBLK1)
        def _():
            pltpu.sync_copy(dst2d_hbm.at[pl.ds(gbase + NBLK1, _R)], idx2d.at[pl.ds(NBLK1, _R)])
            pltpu.sync_copy(sh_hbm.at[pl.ds(gbase + NBLK1, _R)], seh_v.at[pl.ds(NBLK1, _R)])
            pltpu.sync_copy(sl_hbm.at[pl.ds(gbase + NBLK1, _R)], sel_v.at[pl.ds(NBLK1, _R)])

        def issue(g, par):
            idxrow = idx2d.at[g]
            pltpu.async_copy(hcat_hbm.at[idx2d.at[g, pl.ds(0, BE // 2)]], gbuf[par].at[pl.ds(0, BE // 2)], sem_g[par])
            pltpu.async_copy(hcat_hbm.at[idx2d.at[g, pl.ds(BE // 2, BE // 2)]], gbuf[par].at[pl.ds(BE // 2, BE // 2)], sem_g[par])
            pltpu.async_copy(th_hbm.at[idxrow], tvh[par], sem_th[par])
            pltpu.async_copy(tl_hbm.at[idxrow], tvl[par], sem_tl[par])
            pltpu.async_copy(hcat_hbm.at[pl.ds(nbase + g * BN, BN)], obuf[par], sem_o[par])

        issue(0, 0)

        def pair_body(gp, carry):
            for par in range(2):
                g = gp * 2 + par

                @pl.when(g + 1 < nblk)
                def _():
                    issue(g + 1, 1 - par)

                pltpu.make_async_copy(hcat_hbm.at[idx2d.at[g, pl.ds(0, BE // 2)]], gbuf[par].at[pl.ds(0, BE // 2)], sem_g[par]).wait()
                pltpu.make_async_copy(hcat_hbm.at[idx2d.at[g, pl.ds(BE // 2, BE // 2)]], gbuf[par].at[pl.ds(BE // 2, BE // 2)], sem_g[par]).wait()
                pltpu.make_async_copy(th_hbm.at[idx2d.at[g]], tvh[par], sem_th[par]).wait()
                pltpu.make_async_copy(tl_hbm.at[idx2d.at[g]], tvl[par], sem_tl[par]).wait()
                pltpu.make_async_copy(
                    hcat_hbm.at[pl.ds(nbase + g * BN, BN)], obuf[par], sem_o[par]
                ).wait()

                @pl.when(g >= 2)
                def _():
                    pltpu.make_async_copy(
                        aggbuf[par], hacat_hbm.at[pl.ds(nbase + g * BN, BN)], sem_out[par]
                    ).wait()

                def node_body(b, ncarry, par=par, g=g):
                    eoff = b * DEG
                    tv_hv = tvh[par][pl.ds(eoff, DEG)]
                    tv_lv = tvl[par][pl.ds(eoff, DEG)]
                    se_hv = seh_v[g, pl.ds(eoff, DEG)]
                    se_lv = sel_v[g, pl.ds(eoff, DEG)]
                    w_h = jnp.exp(-_leaky(se_hv + tv_hv))
                    w_l = jnp.exp(-_leaky(se_lv + tv_lv))
                    rs_h = jnp.sum(w_h) + 1e-16
                    rs_l = jnp.sum(w_l) + 1e-16
                    wph_v[g, pl.ds(eoff, DEG)] = jnp.minimum(w_h, 6.0) / rs_h
                    wpl_v[g, pl.ds(eoff, DEG)] = jnp.minimum(w_l, 6.0) / rs_l
                    for c in range(GCH):
                        lanes = pl.ds(c * L, L)
                        v = gbuf[par][eoff, lanes]
                        acc_lo, acc_hi = _lohi(v)
                        for j in range(1, DEG):
                            lo, hi = _lohi(gbuf[par][eoff + j, lanes])
                            acc_lo = acc_lo + lo
                            acc_hi = acc_hi + hi
                        own_lo, own_hi = _lohi(obuf[par][b, lanes])
                        if c < GCH // 2:
                            res_lo = 16.0 * own_lo + acc_lo
                            res_hi = 16.0 * own_hi + acc_hi
                        else:
                            res_lo = 16.0 * own_lo - acc_lo
                            res_hi = 16.0 * own_hi - acc_hi
                        aggbuf[par][b, lanes] = _pack_bf(res_lo, res_hi)
                    return ncarry

                lax.fori_loop(0, BN, node_body, 0)
                pltpu.async_copy(
                    aggbuf[par], hacat_hbm.at[pl.ds(nbase + g * BN, BN)], sem_out[par]
                )
            return carry

        lax.fori_loop(0, nblk // 2, pair_body, 0)

        for par in range(2):
            g = nblk - 2 + par
            pltpu.make_async_copy(
                aggbuf[par], hacat_hbm.at[pl.ds(nbase + g * BN, BN)], sem_out[par]
            ).wait()
        pltpu.sync_copy(wph_v.at[pl.ds(0, NBLK1)], wph_hbm.at[pl.ds(gbase, NBLK1)])
        pltpu.sync_copy(wpl_v.at[pl.ds(0, NBLK1)], wpl_hbm.at[pl.ds(gbase, NBLK1)])

        @pl.when(nblk > NBLK1)
        def _():
            pltpu.sync_copy(wph_v.at[pl.ds(NBLK1, _R)], wph_hbm.at[pl.ds(gbase + NBLK1, _R)])
            pltpu.sync_copy(wpl_v.at[pl.ds(NBLK1, _R)], wpl_hbm.at[pl.ds(gbase + NBLK1, _R)])

    return k(hcat, se_h, t_h, se_l, t_l, dst2d)


def _stage3(hacat, wph, wpl, dst2d):
    @functools.partial(
        pl.kernel,
        mesh=_mesh(),
        compiler_params=pltpu.CompilerParams(needs_layout_passes=False),
        out_type=jax.ShapeDtypeStruct((NP, D), jnp.float32),
        scratch_types=[
            pltpu.VMEM((NBLK, BE), jnp.int32),
            pltpu.VMEM((NBLK, BE), jnp.float32),
            pltpu.VMEM((NBLK, BE), jnp.float32),
            pltpu.VMEM((BE, DI), jnp.int32),
            pltpu.VMEM((BE, DI), jnp.int32),
            pltpu.VMEM((BN, D), jnp.float32),
            pltpu.VMEM((BN, D), jnp.float32),
            pltpu.SemaphoreType.DMA,
            pltpu.SemaphoreType.DMA,
            pltpu.SemaphoreType.DMA,
            pltpu.SemaphoreType.DMA,
        ],
    )
    def k(
        hacat_hbm,
        wph_hbm,
        wpl_hbm,
        dst2d_hbm,
        out_hbm,
        idx2d,
        wph_v,
        wpl_v,
        g0,
        g1,
        ob0,
        ob1,
        sg0,
        sg1,
        sout0,
        sout1,
    ):
        gbuf = (g0, g1)
        ob = (ob0, ob1)
        sem_g = (sg0, sg1)
        sem_out = (sout0, sout1)

        nbase, nblk = _worker_span()
        gbase = pl.multiple_of(nbase // BN, 8)
        _R = NBLK0 - NBLK1
        pltpu.sync_copy(dst2d_hbm.at[pl.ds(gbase, NBLK1)], idx2d.at[pl.ds(0, NBLK1)])
        pltpu.sync_copy(wph_hbm.at[pl.ds(gbase, NBLK1)], wph_v.at[pl.ds(0, NBLK1)])
        pltpu.sync_copy(wpl_hbm.at[pl.ds(gbase, NBLK1)], wpl_v.at[pl.ds(0, NBLK1)])

        @pl.when(nblk > NBLK1)
        def _():
            pltpu.sync_copy(dst2d_hbm.at[pl.ds(gbase + NBLK1, _R)], idx2d.at[pl.ds(NBLK1, _R)])
            pltpu.sync_copy(wph_hbm.at[pl.ds(gbase + NBLK1, _R)], wph_v.at[pl.ds(NBLK1, _R)])
            pltpu.sync_copy(wpl_hbm.at[pl.ds(gbase + NBLK1, _R)], wpl_v.at[pl.ds(NBLK1, _R)])

        def issue(g, par):
            pltpu.async_copy(hacat_hbm.at[idx2d.at[g, pl.ds(0, BE // 2)]], gbuf[par].at[pl.ds(0, BE // 2)], sem_g[par])
            pltpu.async_copy(hacat_hbm.at[idx2d.at[g, pl.ds(BE // 2, BE // 2)]], gbuf[par].at[pl.ds(BE // 2, BE // 2)], sem_g[par])

        issue(0, 0)

        def pair_body(gp, carry):
            for par in range(2):
                g = gp * 2 + par

                @pl.when(g + 1 < nblk)
                def _():
                    issue(g + 1, 1 - par)

                pltpu.make_async_copy(hacat_hbm.at[idx2d.at[g, pl.ds(0, BE // 2)]], gbuf[par].at[pl.ds(0, BE // 2)], sem_g[par]).wait()
                pltpu.make_async_copy(hacat_hbm.at[idx2d.at[g, pl.ds(BE // 2, BE // 2)]], gbuf[par].at[pl.ds(BE // 2, BE // 2)], sem_g[par]).wait()

                @pl.when(g >= 2)
                def _():
                    pltpu.make_async_copy(
                        ob[par], out_hbm.at[pl.ds(nbase + g * BN, BN)], sem_out[par]
                    ).wait()

                def node_body(b, ncarry, par=par, g=g):
                    eoff = b * DEG
                    wvh = wph_v[g, pl.ds(eoff, DEG)]
                    wvl = wpl_v[g, pl.ds(eoff, DEG)]
                    for c in range(GCH // 2):
                        zero = jnp.zeros((L,), jnp.float32)
                        a_lh = a_hh = a_ll = a_hl = zero
                        for j in range(DEG):
                            wsh = wvh[j]
                            wsl = wvl[j]
                            vh = gbuf[par][eoff + j, pl.ds(c * L, L)]
                            vl = gbuf[par][eoff + j, pl.ds(DI // 2 + c * L, L)]
                            lo1, hi1 = _lohi(vh)
                            lo2, hi2 = _lohi(vl)
                            a_lh = a_lh + wsh * lo1
                            a_hh = a_hh + wsh * hi1
                            a_ll = a_ll + wsl * lo2
                            a_hl = a_hl + wsl * hi2
                        hp_lo = 0.5 * (a_lh + a_ll)
                        hp_hi = 0.5 * (a_hh + a_hl)
                        y_lo = jnp.minimum(
                            jnp.where(hp_lo > 0, hp_lo, jnp.exp(hp_lo) - 1.0), 6.0
                        )
                        y_hi = jnp.minimum(
                            jnp.where(hp_hi > 0, hp_hi, jnp.exp(hp_hi) - 1.0), 6.0
                        )
                        ob[par][b, pl.ds(c * L, L)] = y_lo
                        ob[par][b, pl.ds(D // 2 + c * L, L)] = y_hi
                    return ncarry

                lax.fori_loop(0, BN, node_body, 0)
                pltpu.async_copy(ob[par], out_hbm.at[pl.ds(nbase + g * BN, BN)], sem_out[par])
            return carry

        lax.fori_loop(0, nblk // 2, pair_body, 0)

        for par in range(2):
            g = nblk - 2 + par
            pltpu.make_async_copy(
                ob[par], out_hbm.at[pl.ds(nbase + g * BN, BN)], sem_out[par]
            ).wait()

    return k(hacat, wph, wpl, dst2d)


def kernel(x, edge_index, W_high, W_low, a_high, a_low):
    dst = edge_index[1].astype(jnp.int32)
    xp = jnp.concatenate([x, jnp.zeros((NP - N, D), jnp.float32)])
    dst2d = jnp.concatenate([dst, jnp.zeros((EP - E,), jnp.int32)]).reshape(NBLKG, BE)
    hcat_i32, s_h, t_h, s_l, t_l = _tc_stage(
        xp, W_high, W_low, a_high[0, :D], a_high[0, D:], a_low[0, :D], a_low[0, D:]
    )
    se_h = jnp.repeat(s_h, DEG).reshape(NBLKG, BE)
    se_l = jnp.repeat(s_l, DEG).reshape(NBLKG, BE)
    hacat, wph, wpl = _stage2(hcat_i32, se_h, t_h, se_l, t_l, dst2d)
    out = _stage3(hacat, wph, wpl, dst2d)
    return out[:N]
